# Initial kernel scaffold; baseline (speedup 1.0000x reference)
import jax
import jax.numpy as jnp
from jax.experimental import pallas as pl


def kernel(src_indices, tgt_indices, src_table, tgt_table):
    raise NotImplementedError("write your pallas kernel here")


# speedup vs baseline: 4.3051x; 4.3051x over previous
import functools

import jax
import jax.numpy as jnp
from jax import lax
from jax.experimental import pallas as pl
from jax.experimental.pallas import tpu as pltpu
from jax.experimental.pallas import tpu_sc as plsc

EMBED = 64
CHUNK = 128
NC, NS = 2, 16
NW = NC * NS


def _make_gather(n_rows: int):
    assert n_rows % (NW * CHUNK) == 0
    rows_per_w = n_rows // NW
    chunks_per_w = rows_per_w // CHUNK
    mesh = plsc.VectorSubcoreMesh(core_axis_name="c", subcore_axis_name="s",
                                  num_cores=NC, num_subcores=NS)

    def body(src_idx, tgt_idx, src_tab, tgt_tab, out_src, out_tgt,
             idx_v, rows_v, sem):
        wid = lax.axis_index("s") * NC + lax.axis_index("c")
        base = wid * rows_per_w
        for idx_hbm, tab, out_hbm in ((src_idx, src_tab, out_src),
                                      (tgt_idx, tgt_tab, out_tgt)):
            pltpu.sync_copy(idx_hbm.at[pl.ds(base, rows_per_w)], idx_v)

            def chunk_body(c, carry, tab=tab, out_hbm=out_hbm):
                pltpu.async_copy(
                    tab.at[idx_v.at[pl.ds(c * CHUNK, CHUNK)]], rows_v,
                    sem).wait()
                pltpu.sync_copy(
                    rows_v, out_hbm.at[pl.ds(base + c * CHUNK, CHUNK)])
                return carry

            lax.fori_loop(0, chunks_per_w, chunk_body, 0)

    out_sd = jax.ShapeDtypeStruct((n_rows, EMBED), jnp.float32)
    return pl.kernel(
        body,
        out_type=(out_sd, out_sd),
        mesh=mesh,
        scratch_types=[
            pltpu.VMEM((rows_per_w,), jnp.int32),
            pltpu.VMEM((CHUNK, EMBED), jnp.float32),
            pltpu.SemaphoreType.DMA,
        ],
        compiler_params=pltpu.CompilerParams(use_tc_tiling_on_sc=False),
    )


def kernel(src_indices, tgt_indices, src_table, tgt_table):
    b, s = src_indices.shape
    n_rows = b * s
    src_flat = src_indices.reshape(n_rows).astype(jnp.int32)
    tgt_flat = tgt_indices.reshape(n_rows).astype(jnp.int32)
    out_src, out_tgt = _make_gather(n_rows)(
        src_flat, tgt_flat, src_table, tgt_table)
    return (out_src.reshape(b, s, EMBED), out_tgt.reshape(b, s, EMBED))


# speedup vs baseline: 4.9650x; 1.1533x over previous
import jax
import jax.numpy as jnp
from jax import lax
from jax.experimental import pallas as pl
from jax.experimental.pallas import tpu as pltpu
from jax.experimental.pallas import tpu_sc as plsc

EMBED = 64
CHUNK = 128
GROUP = 5
GROWS = GROUP * CHUNK
NC, NS = 2, 16
NW = NC * NS


def _make_gather(n_rows: int):
    assert n_rows % (NW * GROWS) == 0
    rows_per_w = n_rows // NW
    n_groups = rows_per_w // GROWS
    assert n_groups % 2 == 0
    mesh = plsc.VectorSubcoreMesh(core_axis_name="c", subcore_axis_name="s",
                                  num_cores=NC, num_subcores=NS)

    def body(src_idx, tgt_idx, src_tab, tgt_tab, out_src, out_tgt,
             idx_v, buf0, buf1, sem):
        wid = lax.axis_index("s") * NC + lax.axis_index("c")
        base = wid * rows_per_w
        bufs = (buf0, buf1)

        for idx_hbm, tab, out_hbm in ((src_idx, src_tab, out_src),
                                      (tgt_idx, tgt_tab, out_tgt)):
            pltpu.sync_copy(idx_hbm.at[pl.ds(base, rows_per_w)], idx_v)

            def fire(g, buf, tab=tab):
                for j in range(GROUP):
                    pltpu.async_copy(
                        tab.at[idx_v.at[pl.ds(g * GROWS + j * CHUNK, CHUNK)]],
                        buf.at[pl.ds(j * CHUNK, CHUNK)], sem)

            def drain(buf, tab=tab):
                pltpu.make_async_copy(tab.at[pl.ds(0, GROWS)], buf, sem).wait()

            fire(0, bufs[0])

            @pl.loop(0, n_groups, step=2)
            def _(g, tab=tab, out_hbm=out_hbm):
                for b in range(2):
                    gg = g + b
                    drain(bufs[b])

                    @pl.when(gg + 1 < n_groups)
                    def _():
                        fire(gg + 1, bufs[1 - b])

                    pltpu.sync_copy(
                        bufs[b], out_hbm.at[pl.ds(base + gg * GROWS, GROWS)])

    out_sd = jax.ShapeDtypeStruct((n_rows, EMBED), jnp.float32)
    return pl.kernel(
        body,
        out_type=(out_sd, out_sd),
        mesh=mesh,
        scratch_types=[
            pltpu.VMEM((rows_per_w,), jnp.int32),
            pltpu.VMEM((GROWS, EMBED), jnp.float32),
            pltpu.VMEM((GROWS, EMBED), jnp.float32),
            pltpu.SemaphoreType.DMA,
        ],
        compiler_params=pltpu.CompilerParams(use_tc_tiling_on_sc=False),
    )


def kernel(src_indices, tgt_indices, src_table, tgt_table):
    b, s = src_indices.shape
    n_rows = b * s
    src_flat = src_indices.reshape(n_rows).astype(jnp.int32)
    tgt_flat = tgt_indices.reshape(n_rows).astype(jnp.int32)
    out_src, out_tgt = _make_gather(n_rows)(
        src_flat, tgt_flat, src_table, tgt_table)
    return (out_src.reshape(b, s, EMBED), out_tgt.reshape(b, s, EMBED))
